# Initial kernel scaffold; baseline (speedup 1.0000x reference)
import jax
import jax.numpy as jnp
from jax.experimental import pallas as pl


def kernel(x, edge, W1, b1, W2, b2):
    raise NotImplementedError("write your pallas kernel here")


# speedup vs baseline: 31.6737x; 31.6737x over previous
import functools

import jax
import jax.numpy as jnp
from jax import lax
from jax.experimental import pallas as pl
from jax.experimental.pallas import tpu as pltpu
from jax.experimental.pallas import tpu_sc as plsc

N = 10000
E = 320000
D_IN = 128
D_HID = 20
D_OUT = 7

NC = 2
NS = 16
NW = NC * NS
CHUNK = 128
NCH = 80
EPAD = NW * NCH * CHUNK
K = 8
NGRP = NCH // K
NPAD = 10112
SLAB = NPAD // NS
DP1 = 24
DP2 = 8

_MESH = plsc.VectorSubcoreMesh(core_axis_name="c", subcore_axis_name="s")
_SC_PARAMS = pltpu.CompilerParams(use_tc_tiling_on_sc=False)


def _slab(sid):
    return pl.ds(sid * SLAB, SLAB)


def _msg_pass_body(dp, table, rowp, colp, zeros, out,
                   rowidx, colidx, zslab, gbuf, acc, gsem, ssem):
    c = lax.axis_index("c")
    s = lax.axis_index("s")
    w = c * NS + s
    pltpu.sync_copy(rowp.at[w], rowidx)
    pltpu.sync_copy(colp.at[w], colidx)
    pltpu.sync_copy(zeros.at[_slab(s)], zslab)
    pltpu.sync_copy(zslab, acc.at[_slab(s)])
    plsc.subcore_barrier()

    @pl.loop(0, NGRP)
    def _grp(g):
        gds = []
        for k in range(K):
            ch = g * K + k
            gds.append(pltpu.async_copy(table.at[rowidx.at[ch]], gbuf.at[k], gsem))
        for gd in gds:
            gd.wait()
        sds = []
        for k in range(K):
            ch = g * K + k
            sds.append(
                pltpu.async_copy(gbuf.at[k], acc.at[colidx.at[ch]], ssem, add=True))
        for sd in sds:
            sd.wait()

    plsc.subcore_barrier()
    pltpu.sync_copy(acc.at[_slab(s)], zslab)
    pltpu.sync_copy(zslab, out.at[c, _slab(s)])


def _make_msg_pass(dp):
    return functools.partial(
        pl.kernel,
        out_type=jax.ShapeDtypeStruct((NC, NPAD, dp), jnp.float32),
        mesh=_MESH,
        scratch_types=[
            pltpu.VMEM((NCH, CHUNK), jnp.int32),
            pltpu.VMEM((NCH, CHUNK), jnp.int32),
            pltpu.VMEM((SLAB, dp), jnp.float32),
            pltpu.VMEM((K, CHUNK, dp), jnp.float32),
            pltpu.VMEM_SHARED((NPAD, dp), jnp.float32),
            pltpu.SemaphoreType.DMA,
            pltpu.SemaphoreType.DMA,
        ],
        compiler_params=_SC_PARAMS,
    )(functools.partial(_msg_pass_body, dp))


_msg_pass1 = _make_msg_pass(DP1)
_msg_pass2 = _make_msg_pass(DP2)


@functools.partial(
    pl.kernel,
    out_type=jax.ShapeDtypeStruct((NC, NPAD, DP2), jnp.float32),
    mesh=_MESH,
    scratch_types=[
        pltpu.VMEM((NCH, CHUNK), jnp.int32),
        pltpu.VMEM((SLAB, DP2), jnp.float32),
        pltpu.VMEM((CHUNK, DP2), jnp.float32),
        pltpu.VMEM_SHARED((NPAD, DP2), jnp.float32),
        pltpu.SemaphoreType.DMA,
    ],
    compiler_params=_SC_PARAMS,
)
def _deg_pass(colp, ones, zeros, out, colidx, zslab, onesbuf, acc, ssem):
    c = lax.axis_index("c")
    s = lax.axis_index("s")
    w = c * NS + s
    pltpu.sync_copy(colp.at[w], colidx)
    pltpu.sync_copy(ones, onesbuf)
    pltpu.sync_copy(zeros.at[_slab(s)], zslab)
    pltpu.sync_copy(zslab, acc.at[_slab(s)])
    plsc.subcore_barrier()

    @pl.loop(0, NGRP)
    def _grp(g):
        sds = []
        for k in range(K):
            ch = g * K + k
            sds.append(
                pltpu.async_copy(onesbuf, acc.at[colidx.at[ch]], ssem, add=True))
        for sd in sds:
            sd.wait()

    plsc.subcore_barrier()
    pltpu.sync_copy(acc.at[_slab(s)], zslab)
    pltpu.sync_copy(zslab, out.at[c, _slab(s)])


_TC_ROWS = 1000
_TC_GRID = N // _TC_ROWS


def _dinv_of(degp):
    deg = degp[0][:, 0:1] + degp[1][:, 0:1] + 1.0
    return lax.rsqrt(deg)


def _tc1_body(x_ref, degp_ref, w1_ref, out_ref):
    h = jnp.dot(x_ref[...], w1_ref[...], preferred_element_type=jnp.float32)
    h1n = h * _dinv_of(degp_ref)
    out_ref[...] = jnp.concatenate(
        [h1n, jnp.zeros((_TC_ROWS, DP1 - D_HID), jnp.float32)], axis=1)


def _tc1(x, degp, W1):
    return pl.pallas_call(
        _tc1_body,
        grid=(_TC_GRID,),
        in_specs=[
            pl.BlockSpec((_TC_ROWS, D_IN), lambda i: (i, 0)),
            pl.BlockSpec((NC, _TC_ROWS, DP2), lambda i: (0, i, 0)),
            pl.BlockSpec((D_IN, D_HID), lambda i: (0, 0)),
        ],
        out_specs=pl.BlockSpec((_TC_ROWS, DP1), lambda i: (i, 0)),
        out_shape=jax.ShapeDtypeStruct((N, DP1), jnp.float32),
    )(x, degp, W1)


def _tc2_body(accp_ref, h1n_ref, degp_ref, b1_ref, w2_ref, out_ref):
    dinv = _dinv_of(degp_ref)
    s1 = (accp_ref[0] + accp_ref[1] + h1n_ref[...])[:, :D_HID]
    out1 = jnp.maximum(dinv * s1 + b1_ref[...], 0.0)
    h2 = jnp.dot(out1, w2_ref[...], preferred_element_type=jnp.float32)
    h2n = h2 * dinv
    out_ref[...] = jnp.concatenate(
        [h2n, jnp.zeros((_TC_ROWS, DP2 - D_OUT), jnp.float32)], axis=1)


def _tc2(accp1, h1n, degp, b1, W2):
    return pl.pallas_call(
        _tc2_body,
        grid=(_TC_GRID,),
        in_specs=[
            pl.BlockSpec((NC, _TC_ROWS, DP1), lambda i: (0, i, 0)),
            pl.BlockSpec((_TC_ROWS, DP1), lambda i: (i, 0)),
            pl.BlockSpec((NC, _TC_ROWS, DP2), lambda i: (0, i, 0)),
            pl.BlockSpec((1, D_HID), lambda i: (0, 0)),
            pl.BlockSpec((D_HID, D_OUT), lambda i: (0, 0)),
        ],
        out_specs=pl.BlockSpec((_TC_ROWS, DP2), lambda i: (i, 0)),
        out_shape=jax.ShapeDtypeStruct((N, DP2), jnp.float32),
    )(accp1, h1n, degp, b1, W2)


def _tc3_body(accp_ref, h2n_ref, degp_ref, b2_ref, out_ref):
    dinv = _dinv_of(degp_ref)
    z = dinv * (accp_ref[0] + accp_ref[1] + h2n_ref[...])[:, :D_OUT] + b2_ref[...]
    m = jnp.max(z, axis=1, keepdims=True)
    lse = jnp.log(jnp.sum(jnp.exp(z - m), axis=1, keepdims=True)) + m
    out_ref[...] = z - lse


def _tc3(accp2, h2n, degp, b2):
    return pl.pallas_call(
        _tc3_body,
        grid=(_TC_GRID,),
        in_specs=[
            pl.BlockSpec((NC, _TC_ROWS, DP2), lambda i: (0, i, 0)),
            pl.BlockSpec((_TC_ROWS, DP2), lambda i: (i, 0)),
            pl.BlockSpec((NC, _TC_ROWS, DP2), lambda i: (0, i, 0)),
            pl.BlockSpec((1, D_OUT), lambda i: (0, 0)),
        ],
        out_specs=pl.BlockSpec((_TC_ROWS, D_OUT), lambda i: (i, 0)),
        out_shape=jax.ShapeDtypeStruct((N, D_OUT), jnp.float32),
    )(accp2, h2n, degp, b2)


def kernel(x, edge, W1, b1, W2, b2):
    row = edge[0].astype(jnp.int32)
    col = edge[1].astype(jnp.int32)
    rowp = jnp.concatenate(
        [row, jnp.zeros((EPAD - E,), jnp.int32)]).reshape(NW, NCH, CHUNK)
    colp = jnp.concatenate(
        [col, jnp.full((EPAD - E,), N, jnp.int32)]).reshape(NW, NCH, CHUNK)
    zeros1 = jnp.zeros((NPAD, DP1), jnp.float32)
    zeros2 = jnp.zeros((NPAD, DP2), jnp.float32)
    ones = jnp.ones((CHUNK, DP2), jnp.float32)

    degp = _deg_pass(colp, ones, zeros2)
    h1n = _tc1(x, degp, W1)
    accp1 = _msg_pass1(h1n, rowp, colp, zeros1)
    h2n = _tc2(accp1, h1n, degp, b1.reshape(1, D_HID), W2)
    accp2 = _msg_pass2(h2n, rowp, colp, zeros2)
    return _tc3(accp2, h2n, degp, b2.reshape(1, D_OUT))


# speedup vs baseline: 34.3630x; 1.0849x over previous
import functools

import jax
import jax.numpy as jnp
from jax import lax
from jax.experimental import pallas as pl
from jax.experimental.pallas import tpu as pltpu
from jax.experimental.pallas import tpu_sc as plsc

N = 10000
E = 320000
D_IN = 128
D_HID = 20
D_OUT = 7

NC = 2
NS = 16
NW = NC * NS
CHUNK = 128
NCH = 80
EPAD = NW * NCH * CHUNK
K = 8
NGRP = NCH // K
NPAD = 10112
SLAB = NPAD // NS
DP1 = 24
DP2 = 8

_MESH = plsc.VectorSubcoreMesh(core_axis_name="c", subcore_axis_name="s")
_SC_PARAMS = pltpu.CompilerParams(use_tc_tiling_on_sc=False)


def _slab(sid):
    return pl.ds(sid * SLAB, SLAB)


NBUF = 16
GAHEAD = 8


def _msg_pass_body(dp, table, rowp, colp, zeros, out,
                   rowidx, colidx, zslab, gbuf, acc, gsem, ssem):
    c = lax.axis_index("c")
    s = lax.axis_index("s")
    w = c * NS + s
    pltpu.sync_copy(rowp.at[w], rowidx)
    pltpu.sync_copy(colp.at[w], colidx)
    pltpu.sync_copy(zeros.at[_slab(s)], zslab)
    pltpu.sync_copy(zslab, acc.at[_slab(s)])
    plsc.subcore_barrier()

    def _gather(ch):
        pltpu.async_copy(table.at[rowidx.at[ch]], gbuf.at[lax.rem(ch, NBUF)], gsem)

    def _gather_wait(ch):
        pltpu.make_async_copy(
            table.at[rowidx.at[ch]], gbuf.at[lax.rem(ch, NBUF)], gsem).wait()

    def _scatter(ch):
        pltpu.async_copy(
            gbuf.at[lax.rem(ch, NBUF)], acc.at[colidx.at[ch]], ssem, add=True)

    def _scatter_wait(ch):
        pltpu.make_async_copy(
            gbuf.at[lax.rem(ch, NBUF)], acc.at[colidx.at[ch]], ssem).wait()

    for ch in range(GAHEAD):
        _gather(ch)

    @pl.loop(0, NCH)
    def _step(j):
        @pl.when(j >= GAHEAD)
        def _():
            _scatter_wait(j - GAHEAD)

        @pl.when(j + GAHEAD < NCH)
        def _():
            _gather(j + GAHEAD)

        _gather_wait(j)
        _scatter(j)

    @pl.loop(NCH - GAHEAD, NCH)
    def _drain(j):
        _scatter_wait(j)

    plsc.subcore_barrier()
    pltpu.sync_copy(acc.at[_slab(s)], zslab)
    pltpu.sync_copy(zslab, out.at[c, _slab(s)])


def _make_msg_pass(dp):
    return functools.partial(
        pl.kernel,
        out_type=jax.ShapeDtypeStruct((NC, NPAD, dp), jnp.float32),
        mesh=_MESH,
        scratch_types=[
            pltpu.VMEM((NCH, CHUNK), jnp.int32),
            pltpu.VMEM((NCH, CHUNK), jnp.int32),
            pltpu.VMEM((SLAB, dp), jnp.float32),
            pltpu.VMEM((NBUF, CHUNK, dp), jnp.float32),
            pltpu.VMEM_SHARED((NPAD, dp), jnp.float32),
            pltpu.SemaphoreType.DMA,
            pltpu.SemaphoreType.DMA,
        ],
        compiler_params=_SC_PARAMS,
    )(functools.partial(_msg_pass_body, dp))


_msg_pass1 = _make_msg_pass(DP1)
_msg_pass2 = _make_msg_pass(DP2)


@functools.partial(
    pl.kernel,
    out_type=jax.ShapeDtypeStruct((NC, NPAD, DP2), jnp.float32),
    mesh=_MESH,
    scratch_types=[
        pltpu.VMEM((NCH, CHUNK), jnp.int32),
        pltpu.VMEM((SLAB, DP2), jnp.float32),
        pltpu.VMEM((CHUNK, DP2), jnp.float32),
        pltpu.VMEM_SHARED((NPAD, DP2), jnp.float32),
        pltpu.SemaphoreType.DMA,
    ],
    compiler_params=_SC_PARAMS,
)
def _deg_pass(colp, ones, zeros, out, colidx, zslab, onesbuf, acc, ssem):
    c = lax.axis_index("c")
    s = lax.axis_index("s")
    w = c * NS + s
    pltpu.sync_copy(colp.at[w], colidx)
    pltpu.sync_copy(ones, onesbuf)
    pltpu.sync_copy(zeros.at[_slab(s)], zslab)
    pltpu.sync_copy(zslab, acc.at[_slab(s)])
    plsc.subcore_barrier()

    @pl.loop(0, NCH)
    def _fire(j):
        pltpu.async_copy(onesbuf, acc.at[colidx.at[j]], ssem, add=True)

    @pl.loop(0, NCH)
    def _drain(j):
        pltpu.make_async_copy(onesbuf, acc.at[colidx.at[j]], ssem).wait()

    plsc.subcore_barrier()
    pltpu.sync_copy(acc.at[_slab(s)], zslab)
    pltpu.sync_copy(zslab, out.at[c, _slab(s)])


_TC_ROWS = 1000
_TC_GRID = N // _TC_ROWS


def _dinv_of(degp):
    deg = degp[0][:, 0:1] + degp[1][:, 0:1] + 1.0
    return lax.rsqrt(deg)


def _tc1_body(x_ref, degp_ref, w1_ref, out_ref):
    h = jnp.dot(x_ref[...], w1_ref[...], preferred_element_type=jnp.float32)
    h1n = h * _dinv_of(degp_ref)
    out_ref[...] = jnp.concatenate(
        [h1n, jnp.zeros((_TC_ROWS, DP1 - D_HID), jnp.float32)], axis=1)


def _tc1(x, degp, W1):
    return pl.pallas_call(
        _tc1_body,
        grid=(_TC_GRID,),
        in_specs=[
            pl.BlockSpec((_TC_ROWS, D_IN), lambda i: (i, 0)),
            pl.BlockSpec((NC, _TC_ROWS, DP2), lambda i: (0, i, 0)),
            pl.BlockSpec((D_IN, D_HID), lambda i: (0, 0)),
        ],
        out_specs=pl.BlockSpec((_TC_ROWS, DP1), lambda i: (i, 0)),
        out_shape=jax.ShapeDtypeStruct((N, DP1), jnp.float32),
    )(x, degp, W1)


def _tc2_body(accp_ref, h1n_ref, degp_ref, b1_ref, w2_ref, out_ref):
    dinv = _dinv_of(degp_ref)
    s1 = (accp_ref[0] + accp_ref[1] + h1n_ref[...])[:, :D_HID]
    out1 = jnp.maximum(dinv * s1 + b1_ref[...], 0.0)
    h2 = jnp.dot(out1, w2_ref[...], preferred_element_type=jnp.float32)
    h2n = h2 * dinv
    out_ref[...] = jnp.concatenate(
        [h2n, jnp.zeros((_TC_ROWS, DP2 - D_OUT), jnp.float32)], axis=1)


def _tc2(accp1, h1n, degp, b1, W2):
    return pl.pallas_call(
        _tc2_body,
        grid=(_TC_GRID,),
        in_specs=[
            pl.BlockSpec((NC, _TC_ROWS, DP1), lambda i: (0, i, 0)),
            pl.BlockSpec((_TC_ROWS, DP1), lambda i: (i, 0)),
            pl.BlockSpec((NC, _TC_ROWS, DP2), lambda i: (0, i, 0)),
            pl.BlockSpec((1, D_HID), lambda i: (0, 0)),
            pl.BlockSpec((D_HID, D_OUT), lambda i: (0, 0)),
        ],
        out_specs=pl.BlockSpec((_TC_ROWS, DP2), lambda i: (i, 0)),
        out_shape=jax.ShapeDtypeStruct((N, DP2), jnp.float32),
    )(accp1, h1n, degp, b1, W2)


def _tc3_body(accp_ref, h2n_ref, degp_ref, b2_ref, out_ref):
    dinv = _dinv_of(degp_ref)
    z = dinv * (accp_ref[0] + accp_ref[1] + h2n_ref[...])[:, :D_OUT] + b2_ref[...]
    m = jnp.max(z, axis=1, keepdims=True)
    lse = jnp.log(jnp.sum(jnp.exp(z - m), axis=1, keepdims=True)) + m
    out_ref[...] = z - lse


def _tc3(accp2, h2n, degp, b2):
    return pl.pallas_call(
        _tc3_body,
        grid=(_TC_GRID,),
        in_specs=[
            pl.BlockSpec((NC, _TC_ROWS, DP2), lambda i: (0, i, 0)),
            pl.BlockSpec((_TC_ROWS, DP2), lambda i: (i, 0)),
            pl.BlockSpec((NC, _TC_ROWS, DP2), lambda i: (0, i, 0)),
            pl.BlockSpec((1, D_OUT), lambda i: (0, 0)),
        ],
        out_specs=pl.BlockSpec((_TC_ROWS, D_OUT), lambda i: (i, 0)),
        out_shape=jax.ShapeDtypeStruct((N, D_OUT), jnp.float32),
    )(accp2, h2n, degp, b2)


def kernel(x, edge, W1, b1, W2, b2):
    row = edge[0].astype(jnp.int32)
    col = edge[1].astype(jnp.int32)
    rowp = jnp.concatenate(
        [row, jnp.zeros((EPAD - E,), jnp.int32)]).reshape(NW, NCH, CHUNK)
    colp = jnp.concatenate(
        [col, jnp.full((EPAD - E,), N, jnp.int32)]).reshape(NW, NCH, CHUNK)
    zeros1 = jnp.zeros((NPAD, DP1), jnp.float32)
    zeros2 = jnp.zeros((NPAD, DP2), jnp.float32)
    ones = jnp.ones((CHUNK, DP2), jnp.float32)

    degp = _deg_pass(colp, ones, zeros2)
    h1n = _tc1(x, degp, W1)
    accp1 = _msg_pass1(h1n, rowp, colp, zeros1)
    h2n = _tc2(accp1, h1n, degp, b1.reshape(1, D_HID), W2)
    accp2 = _msg_pass2(h2n, rowp, colp, zeros2)
    return _tc3(accp2, h2n, degp, b2.reshape(1, D_OUT))


# speedup vs baseline: 53.3586x; 1.5528x over previous
import functools

import jax
import jax.numpy as jnp
from jax import lax
from jax.experimental import pallas as pl
from jax.experimental.pallas import tpu as pltpu
from jax.experimental.pallas import tpu_sc as plsc

N = 10000
E = 320000
D_IN = 128
D_HID = 20
D_OUT = 7

NC = 2
NS = 16
NW = NC * NS
CHUNK = 128
NCH = 80
EPAD = NW * NCH * CHUNK
K = 8
NGRP = NCH // K
NPAD = 10112
SLAB = NPAD // NS
DP1 = 24
DP2 = 8

_MESH = plsc.VectorSubcoreMesh(core_axis_name="c", subcore_axis_name="s")
_SC_PARAMS = pltpu.CompilerParams(use_tc_tiling_on_sc=False)


def _slab(sid):
    return pl.ds(sid * SLAB, SLAB)


NBUF = 16
GAHEAD = 8


TSLAB = N // NS


def _msg_pass_body(dp, table, rowp, colp, zeros, out,
                   rowidx, colidx, zslab, tslab, gbuf, tbl, acc, gsem, ssem):
    c = lax.axis_index("c")
    s = lax.axis_index("s")
    w = c * NS + s
    pltpu.sync_copy(rowp.at[w], rowidx)
    pltpu.sync_copy(colp.at[w], colidx)
    tsl = pl.ds(s * TSLAB, TSLAB)
    pltpu.sync_copy(table.at[tsl], tslab)
    pltpu.sync_copy(tslab, tbl.at[tsl])
    pltpu.sync_copy(zeros.at[_slab(s)], zslab)
    pltpu.sync_copy(zslab, acc.at[_slab(s)])
    plsc.subcore_barrier()

    def _gather(ch):
        pltpu.async_copy(tbl.at[rowidx.at[ch]], gbuf.at[lax.rem(ch, NBUF)], gsem)

    def _gather_wait(ch):
        pltpu.make_async_copy(
            tbl.at[rowidx.at[ch]], gbuf.at[lax.rem(ch, NBUF)], gsem).wait()

    def _scatter(ch):
        pltpu.async_copy(
            gbuf.at[lax.rem(ch, NBUF)], acc.at[colidx.at[ch]], ssem, add=True)

    def _scatter_wait(ch):
        pltpu.make_async_copy(
            gbuf.at[lax.rem(ch, NBUF)], acc.at[colidx.at[ch]], ssem).wait()

    for ch in range(GAHEAD):
        _gather(ch)

    @pl.loop(0, NCH)
    def _step(j):
        @pl.when(j >= GAHEAD)
        def _():
            _scatter_wait(j - GAHEAD)

        @pl.when(j + GAHEAD < NCH)
        def _():
            _gather(j + GAHEAD)

        _gather_wait(j)
        _scatter(j)

    @pl.loop(NCH - GAHEAD, NCH)
    def _drain(j):
        _scatter_wait(j)

    plsc.subcore_barrier()
    pltpu.sync_copy(acc.at[_slab(s)], zslab)
    pltpu.sync_copy(zslab, out.at[c, _slab(s)])


def _make_msg_pass(dp):
    return functools.partial(
        pl.kernel,
        out_type=jax.ShapeDtypeStruct((NC, NPAD, dp), jnp.float32),
        mesh=_MESH,
        scratch_types=[
            pltpu.VMEM((NCH, CHUNK), jnp.int32),
            pltpu.VMEM((NCH, CHUNK), jnp.int32),
            pltpu.VMEM((SLAB, dp), jnp.float32),
            pltpu.VMEM((TSLAB, dp), jnp.float32),
            pltpu.VMEM((NBUF, CHUNK, dp), jnp.float32),
            pltpu.VMEM_SHARED((N, dp), jnp.float32),
            pltpu.VMEM_SHARED((NPAD, dp), jnp.float32),
            pltpu.SemaphoreType.DMA,
            pltpu.SemaphoreType.DMA,
        ],
        compiler_params=_SC_PARAMS,
    )(functools.partial(_msg_pass_body, dp))


_msg_pass1 = _make_msg_pass(DP1)
_msg_pass2 = _make_msg_pass(DP2)


@functools.partial(
    pl.kernel,
    out_type=jax.ShapeDtypeStruct((NC, NPAD, DP2), jnp.float32),
    mesh=_MESH,
    scratch_types=[
        pltpu.VMEM((NCH, CHUNK), jnp.int32),
        pltpu.VMEM((SLAB, DP2), jnp.float32),
        pltpu.VMEM((CHUNK, DP2), jnp.float32),
        pltpu.VMEM_SHARED((NPAD, DP2), jnp.float32),
        pltpu.SemaphoreType.DMA,
    ],
    compiler_params=_SC_PARAMS,
)
def _deg_pass(colp, ones, zeros, out, colidx, zslab, onesbuf, acc, ssem):
    c = lax.axis_index("c")
    s = lax.axis_index("s")
    w = c * NS + s
    pltpu.sync_copy(colp.at[w], colidx)
    pltpu.sync_copy(ones, onesbuf)
    pltpu.sync_copy(zeros.at[_slab(s)], zslab)
    pltpu.sync_copy(zslab, acc.at[_slab(s)])
    plsc.subcore_barrier()

    @pl.loop(0, NCH)
    def _fire(j):
        pltpu.async_copy(onesbuf, acc.at[colidx.at[j]], ssem, add=True)

    @pl.loop(0, NCH)
    def _drain(j):
        pltpu.make_async_copy(onesbuf, acc.at[colidx.at[j]], ssem).wait()

    plsc.subcore_barrier()
    pltpu.sync_copy(acc.at[_slab(s)], zslab)
    pltpu.sync_copy(zslab, out.at[c, _slab(s)])


_TC_ROWS = 1000
_TC_GRID = N // _TC_ROWS


def _dinv_of(degp):
    deg = degp[0][:, 0:1] + degp[1][:, 0:1] + 1.0
    return lax.rsqrt(deg)


def _tc1_body(x_ref, degp_ref, w1_ref, out_ref):
    h = jnp.dot(x_ref[...], w1_ref[...], preferred_element_type=jnp.float32)
    h1n = h * _dinv_of(degp_ref)
    out_ref[...] = jnp.concatenate(
        [h1n, jnp.zeros((_TC_ROWS, DP1 - D_HID), jnp.float32)], axis=1)


def _tc1(x, degp, W1):
    return pl.pallas_call(
        _tc1_body,
        grid=(_TC_GRID,),
        in_specs=[
            pl.BlockSpec((_TC_ROWS, D_IN), lambda i: (i, 0)),
            pl.BlockSpec((NC, _TC_ROWS, DP2), lambda i: (0, i, 0)),
            pl.BlockSpec((D_IN, D_HID), lambda i: (0, 0)),
        ],
        out_specs=pl.BlockSpec((_TC_ROWS, DP1), lambda i: (i, 0)),
        out_shape=jax.ShapeDtypeStruct((N, DP1), jnp.float32),
    )(x, degp, W1)


def _tc2_body(accp_ref, h1n_ref, degp_ref, b1_ref, w2_ref, out_ref):
    dinv = _dinv_of(degp_ref)
    s1 = (accp_ref[0] + accp_ref[1] + h1n_ref[...])[:, :D_HID]
    out1 = jnp.maximum(dinv * s1 + b1_ref[...], 0.0)
    h2 = jnp.dot(out1, w2_ref[...], preferred_element_type=jnp.float32)
    h2n = h2 * dinv
    out_ref[...] = jnp.concatenate(
        [h2n, jnp.zeros((_TC_ROWS, DP2 - D_OUT), jnp.float32)], axis=1)


def _tc2(accp1, h1n, degp, b1, W2):
    return pl.pallas_call(
        _tc2_body,
        grid=(_TC_GRID,),
        in_specs=[
            pl.BlockSpec((NC, _TC_ROWS, DP1), lambda i: (0, i, 0)),
            pl.BlockSpec((_TC_ROWS, DP1), lambda i: (i, 0)),
            pl.BlockSpec((NC, _TC_ROWS, DP2), lambda i: (0, i, 0)),
            pl.BlockSpec((1, D_HID), lambda i: (0, 0)),
            pl.BlockSpec((D_HID, D_OUT), lambda i: (0, 0)),
        ],
        out_specs=pl.BlockSpec((_TC_ROWS, DP2), lambda i: (i, 0)),
        out_shape=jax.ShapeDtypeStruct((N, DP2), jnp.float32),
    )(accp1, h1n, degp, b1, W2)


def _tc3_body(accp_ref, h2n_ref, degp_ref, b2_ref, out_ref):
    dinv = _dinv_of(degp_ref)
    z = dinv * (accp_ref[0] + accp_ref[1] + h2n_ref[...])[:, :D_OUT] + b2_ref[...]
    m = jnp.max(z, axis=1, keepdims=True)
    lse = jnp.log(jnp.sum(jnp.exp(z - m), axis=1, keepdims=True)) + m
    out_ref[...] = z - lse


def _tc3(accp2, h2n, degp, b2):
    return pl.pallas_call(
        _tc3_body,
        grid=(_TC_GRID,),
        in_specs=[
            pl.BlockSpec((NC, _TC_ROWS, DP2), lambda i: (0, i, 0)),
            pl.BlockSpec((_TC_ROWS, DP2), lambda i: (i, 0)),
            pl.BlockSpec((NC, _TC_ROWS, DP2), lambda i: (0, i, 0)),
            pl.BlockSpec((1, D_OUT), lambda i: (0, 0)),
        ],
        out_specs=pl.BlockSpec((_TC_ROWS, D_OUT), lambda i: (i, 0)),
        out_shape=jax.ShapeDtypeStruct((N, D_OUT), jnp.float32),
    )(accp2, h2n, degp, b2)


def kernel(x, edge, W1, b1, W2, b2):
    row = edge[0].astype(jnp.int32)
    col = edge[1].astype(jnp.int32)
    rowp = jnp.concatenate(
        [row, jnp.zeros((EPAD - E,), jnp.int32)]).reshape(NW, NCH, CHUNK)
    colp = jnp.concatenate(
        [col, jnp.full((EPAD - E,), N, jnp.int32)]).reshape(NW, NCH, CHUNK)
    zeros1 = jnp.zeros((NPAD, DP1), jnp.float32)
    zeros2 = jnp.zeros((NPAD, DP2), jnp.float32)
    ones = jnp.ones((CHUNK, DP2), jnp.float32)

    degp = _deg_pass(colp, ones, zeros2)
    h1n = _tc1(x, degp, W1)
    accp1 = _msg_pass1(h1n, rowp, colp, zeros1)
    h2n = _tc2(accp1, h1n, degp, b1.reshape(1, D_HID), W2)
    accp2 = _msg_pass2(h2n, rowp, colp, zeros2)
    return _tc3(accp2, h2n, degp, b2.reshape(1, D_OUT))
